# baseline (speedup 1.0000x reference)
import jax
import jax.numpy as jnp
from jax.experimental import pallas as pl
from jax.experimental.pallas import tpu as pltpu


def _round_up(a: int, b: int) -> int:
    return ((a + b - 1) // b) * b


def _xw_kernel(x_ref, w_ref, s_ref):
    s_ref[...] = jnp.dot(
        x_ref[...].astype(jnp.bfloat16),
        w_ref[...].astype(jnp.bfloat16),
        preferred_element_type=jnp.float32,
    ).astype(s_ref.dtype)


def _prop_kernel(adj_a_ref, adj_b_ref, s_a_ref, s_b_ref, b_ref, out_ref):
    acc = jnp.dot(
        adj_a_ref[...].astype(jnp.bfloat16),
        s_a_ref[...],
        preferred_element_type=jnp.float32,
    )
    acc += jnp.dot(
        adj_b_ref[...].astype(jnp.bfloat16),
        s_b_ref[...],
        preferred_element_type=jnp.float32,
    )
    out_ref[...] = (acc + b_ref[...]).astype(out_ref.dtype)


def kernel(x, adj, weight, bias):
    N, f_in = x.shape
    f_in_w, f_out = weight.shape
    assert f_in == f_in_w, "weight shape mismatch"
    assert adj.shape == (N, N), "adj must be [N, N]"

    out_dtype = x.dtype

    N_pad = _round_up(N, 256)
    f_out_pad = _round_up(f_out, 128)

    x_p = jnp.pad(x, ((0, N_pad - N), (0, 0)))
    adj_p = jnp.pad(adj, ((0, N_pad - N), (0, N_pad - N)))
    w_p = jnp.pad(weight, ((0, 0), (0, f_out_pad - f_out)))
    b = bias if bias is not None else jnp.zeros((f_out,), out_dtype)
    b_p = jnp.pad(b, (0, f_out_pad - f_out)).reshape(1, f_out_pad).astype(jnp.float32)

    tm1 = 512 if N_pad % 512 == 0 else 256
    tm2 = 1024

    support = pl.pallas_call(
        _xw_kernel,
        out_shape=jax.ShapeDtypeStruct((N_pad, f_out_pad), jnp.bfloat16),
        grid=(N_pad // tm1,),
        in_specs=[
            pl.BlockSpec((tm1, f_in), lambda i: (i, 0)),
            pl.BlockSpec((f_in, f_out_pad), lambda i: (0, 0)),
        ],
        out_specs=pl.BlockSpec((tm1, f_out_pad), lambda i: (i, 0)),
        compiler_params=pltpu.CompilerParams(
            dimension_semantics=("parallel",),
        ),
    )(x_p, w_p)

    kh = N_pad // 2
    out_p = pl.pallas_call(
        _prop_kernel,
        out_shape=jax.ShapeDtypeStruct((N_pad, f_out_pad), out_dtype),
        grid=(N_pad // tm2,),
        in_specs=[
            pl.BlockSpec((tm2, kh), lambda i: (i, 0)),
            pl.BlockSpec((tm2, kh), lambda i: (i, 1)),
            pl.BlockSpec((kh, f_out_pad), lambda i: (0, 0)),
            pl.BlockSpec((kh, f_out_pad), lambda i: (1, 0)),
            pl.BlockSpec((1, f_out_pad), lambda i: (0, 0)),
        ],
        out_specs=pl.BlockSpec((tm2, f_out_pad), lambda i: (i, 0)),
        compiler_params=pltpu.CompilerParams(
            dimension_semantics=("parallel",),
        ),
    )(adj_p, adj_p, support, support, b_p)

    return out_p[:N, :f_out]


# speedup vs baseline: 1.0156x; 1.0156x over previous
import jax
import jax.numpy as jnp
from jax.experimental import pallas as pl
from jax.experimental.pallas import tpu as pltpu


def _round_up(a: int, b: int) -> int:
    return ((a + b - 1) // b) * b


def _xw_kernel(x_ref, w_ref, s_ref):
    s_ref[...] = jnp.dot(
        x_ref[...].astype(jnp.bfloat16),
        w_ref[...].astype(jnp.bfloat16),
        preferred_element_type=jnp.float32,
    ).astype(s_ref.dtype)


def _make_prop_kernel(n_s):
    def _prop_kernel(*refs):
        adj_refs = refs[:n_s]
        s_refs = refs[n_s : 2 * n_s]
        b_ref = refs[2 * n_s]
        out_ref = refs[2 * n_s + 1]
        acc = jnp.dot(
            adj_refs[0][...].astype(jnp.bfloat16),
            s_refs[0][...],
            preferred_element_type=jnp.float32,
        )
        for j in range(1, n_s):
            acc += jnp.dot(
                adj_refs[j][...].astype(jnp.bfloat16),
                s_refs[j][...],
                preferred_element_type=jnp.float32,
            )
        out_ref[...] = (acc + b_ref[...]).astype(out_ref.dtype)

    return _prop_kernel


def kernel(x, adj, weight, bias):
    N, f_in = x.shape
    f_in_w, f_out = weight.shape
    assert f_in == f_in_w, "weight shape mismatch"
    assert adj.shape == (N, N), "adj must be [N, N]"

    out_dtype = x.dtype

    N_pad = _round_up(N, 256)
    f_out_pad = _round_up(f_out, 128)

    x_p = jnp.pad(x, ((0, N_pad - N), (0, 0)))
    adj_p = jnp.pad(adj, ((0, N_pad - N), (0, N_pad - N)))
    w_p = jnp.pad(weight, ((0, 0), (0, f_out_pad - f_out)))
    b = bias if bias is not None else jnp.zeros((f_out,), out_dtype)
    b_p = jnp.pad(b, (0, f_out_pad - f_out)).reshape(1, f_out_pad).astype(jnp.float32)

    tm1 = 512 if N_pad % 512 == 0 else 256
    tm2 = 1024

    support = pl.pallas_call(
        _xw_kernel,
        out_shape=jax.ShapeDtypeStruct((N_pad, f_out_pad), jnp.bfloat16),
        grid=(N_pad // tm1,),
        in_specs=[
            pl.BlockSpec((tm1, f_in), lambda i: (i, 0)),
            pl.BlockSpec((f_in, f_out_pad), lambda i: (0, 0)),
        ],
        out_specs=pl.BlockSpec((tm1, f_out_pad), lambda i: (i, 0)),
        compiler_params=pltpu.CompilerParams(
            dimension_semantics=("parallel",),
        ),
    )(x_p, w_p)

    n_s = 4
    kh = N_pad // n_s
    adj_specs = [
        pl.BlockSpec((tm2, kh), lambda i, j=j: (i, j)) for j in range(n_s)
    ]
    s_specs = [
        pl.BlockSpec((kh, f_out_pad), lambda i, j=j: (j, 0)) for j in range(n_s)
    ]
    out_p = pl.pallas_call(
        _make_prop_kernel(n_s),
        out_shape=jax.ShapeDtypeStruct((N_pad, f_out_pad), out_dtype),
        grid=(N_pad // tm2,),
        in_specs=adj_specs
        + s_specs
        + [pl.BlockSpec((1, f_out_pad), lambda i: (0, 0))],
        out_specs=pl.BlockSpec((tm2, f_out_pad), lambda i: (i, 0)),
        compiler_params=pltpu.CompilerParams(
            dimension_semantics=("parallel",),
        ),
    )(*([adj_p] * n_s + [support] * n_s + [b_p]))

    return out_p[:N, :f_out]


# speedup vs baseline: 1.0435x; 1.0275x over previous
import jax
import jax.numpy as jnp
from jax.experimental import pallas as pl
from jax.experimental.pallas import tpu as pltpu


def _round_up(a: int, b: int) -> int:
    return ((a + b - 1) // b) * b


def _xw_kernel(x_ref, w_ref, s_ref):
    s_ref[...] = jnp.dot(
        x_ref[...].astype(jnp.bfloat16),
        w_ref[...].astype(jnp.bfloat16),
        preferred_element_type=jnp.float32,
    ).astype(s_ref.dtype)


def _make_prop_kernel(n_s):
    def _prop_kernel(*refs):
        adj_refs = refs[:n_s]
        s_refs = refs[n_s : 2 * n_s]
        b_ref = refs[2 * n_s]
        out_ref = refs[2 * n_s + 1]
        acc = jnp.dot(
            adj_refs[0][...].astype(jnp.bfloat16),
            s_refs[0][...],
            preferred_element_type=jnp.float32,
        )
        for j in range(1, n_s):
            acc += jnp.dot(
                adj_refs[j][...].astype(jnp.bfloat16),
                s_refs[j][...],
                preferred_element_type=jnp.float32,
            )
        out_ref[...] = (acc + b_ref[...]).astype(out_ref.dtype)

    return _prop_kernel


def kernel(x, adj, weight, bias):
    N, f_in = x.shape
    f_in_w, f_out = weight.shape
    assert f_in == f_in_w, "weight shape mismatch"
    assert adj.shape == (N, N), "adj must be [N, N]"

    out_dtype = x.dtype

    N_pad = _round_up(N, 256)
    f_out_pad = _round_up(f_out, 128)

    x_p = jnp.pad(x, ((0, N_pad - N), (0, 0)))
    adj_p = jnp.pad(adj, ((0, N_pad - N), (0, N_pad - N)))
    w_p = jnp.pad(weight, ((0, 0), (0, f_out_pad - f_out)))
    b = bias if bias is not None else jnp.zeros((f_out,), out_dtype)
    b_p = jnp.pad(b, (0, f_out_pad - f_out)).reshape(1, f_out_pad).astype(jnp.float32)

    tm1 = 512 if N_pad % 512 == 0 else 256
    tm2 = 512

    support = pl.pallas_call(
        _xw_kernel,
        out_shape=jax.ShapeDtypeStruct((N_pad, f_out_pad), jnp.bfloat16),
        grid=(N_pad // tm1,),
        in_specs=[
            pl.BlockSpec((tm1, f_in), lambda i: (i, 0)),
            pl.BlockSpec((f_in, f_out_pad), lambda i: (0, 0)),
        ],
        out_specs=pl.BlockSpec((tm1, f_out_pad), lambda i: (i, 0)),
        compiler_params=pltpu.CompilerParams(
            dimension_semantics=("parallel",),
        ),
    )(x_p, w_p)

    n_s = 4
    kh = N_pad // n_s
    adj_specs = [
        pl.BlockSpec((tm2, kh), lambda i, j=j: (i, j)) for j in range(n_s)
    ]
    s_specs = [
        pl.BlockSpec((kh, f_out_pad), lambda i, j=j: (j, 0)) for j in range(n_s)
    ]
    out_p = pl.pallas_call(
        _make_prop_kernel(n_s),
        out_shape=jax.ShapeDtypeStruct((N_pad, f_out_pad), out_dtype),
        grid=(N_pad // tm2,),
        in_specs=adj_specs
        + s_specs
        + [pl.BlockSpec((1, f_out_pad), lambda i: (0, 0))],
        out_specs=pl.BlockSpec((tm2, f_out_pad), lambda i: (i, 0)),
        compiler_params=pltpu.CompilerParams(
            dimension_semantics=("parallel",),
        ),
    )(*([adj_p] * n_s + [support] * n_s + [b_p]))

    return out_p[:N, :f_out]


# speedup vs baseline: 1.2519x; 1.1997x over previous
import jax
import jax.numpy as jnp
from jax.experimental import pallas as pl
from jax.experimental.pallas import tpu as pltpu


def _round_up(a: int, b: int) -> int:
    return ((a + b - 1) // b) * b


def _make_fused_kernel(kh):
    def _fused_kernel(x_ref, w_ref, adj_a_ref, adj_b_ref, b_ref, out_ref, s_ref):
        i = pl.program_id(1)

        @pl.when(i == 0)
        def _():
            s_ref[...] = jnp.dot(
                x_ref[...].astype(jnp.bfloat16),
                w_ref[...].astype(jnp.bfloat16),
                preferred_element_type=jnp.float32,
            ).astype(s_ref.dtype)

        acc = jnp.dot(
            adj_a_ref[...].astype(jnp.bfloat16),
            s_ref[:kh, :],
            preferred_element_type=jnp.float32,
        )
        acc += jnp.dot(
            adj_b_ref[...].astype(jnp.bfloat16),
            s_ref[kh:, :],
            preferred_element_type=jnp.float32,
        )
        out_ref[...] = (acc + b_ref[...]).astype(out_ref.dtype)

    return _fused_kernel


def kernel(x, adj, weight, bias):
    N, f_in = x.shape
    f_in_w, f_out = weight.shape
    assert f_in == f_in_w, "weight shape mismatch"
    assert adj.shape == (N, N), "adj must be [N, N]"

    out_dtype = x.dtype

    N_pad = _round_up(N, 256)
    f_out_pad = _round_up(f_out, 128)

    x_p = jnp.pad(x, ((0, N_pad - N), (0, 0)))
    adj_p = jnp.pad(adj, ((0, N_pad - N), (0, N_pad - N)))
    w_p = jnp.pad(weight, ((0, 0), (0, f_out_pad - f_out)))
    b = bias if bias is not None else jnp.zeros((f_out,), out_dtype)
    b_p = jnp.pad(b, (0, f_out_pad - f_out)).reshape(1, f_out_pad).astype(jnp.float32)

    tm = 512 if N_pad % 1024 == 0 else 256
    kh = N_pad // 2
    n_rows = N_pad // tm
    g_in = n_rows // 2

    out_p = pl.pallas_call(
        _make_fused_kernel(kh),
        out_shape=jax.ShapeDtypeStruct((N_pad, f_out_pad), out_dtype),
        grid=(2, g_in),
        in_specs=[
            pl.BlockSpec((N_pad, f_in), lambda o, i: (0, 0)),
            pl.BlockSpec((f_in, f_out_pad), lambda o, i: (0, 0)),
            pl.BlockSpec((tm, kh), lambda o, i: (o * g_in + i, 0)),
            pl.BlockSpec((tm, kh), lambda o, i: (o * g_in + i, 1)),
            pl.BlockSpec((1, f_out_pad), lambda o, i: (0, 0)),
        ],
        out_specs=pl.BlockSpec((tm, f_out_pad), lambda o, i: (o * g_in + i, 0)),
        scratch_shapes=[pltpu.VMEM((N_pad, f_out_pad), jnp.bfloat16)],
        compiler_params=pltpu.CompilerParams(
            dimension_semantics=("parallel", "arbitrary"),
        ),
    )(x_p, w_p, adj_p, adj_p, b_p)

    return out_p[:N, :f_out]
